# baseline (speedup 1.0000x reference)
import jax
import jax.numpy as jnp
from jax import lax
from jax.experimental import pallas as pl
from jax.experimental.pallas import tpu as pltpu
from jax.experimental.pallas import tpu_sc as plsc

V = 100000
D = 64
B = 1024
C = 200

NC = 2
NS = 16
NW = NC * NS
NB = B // NW
IDX_PER_W = NB * C

TV = 2048
EPS = 1e-8


def _query_body(ctx_hbm, miss_hbm, table_hbm, out_hbm,
                idx_v, rows_v, midx_v, mrows_v, q_v, gsem, msem):
    c = lax.axis_index("c")
    s = lax.axis_index("s")
    w = s * NC + c
    base = w * NB

    pltpu.sync_copy(ctx_hbm.at[pl.ds(base * C, IDX_PER_W)], idx_v)
    pltpu.sync_copy(miss_hbm.at[pl.ds(base, NB)], midx_v)
    pltpu.async_copy(table_hbm.at[midx_v], mrows_v, msem).wait()

    inv_c = jnp.float32(1.0 / C)

    def item(i, carry):
        cp0 = pltpu.async_copy(table_hbm.at[idx_v.at[pl.ds(i * C, 128)]],
                               rows_v.at[pl.ds(0, 128)], gsem)
        cp1 = pltpu.async_copy(table_hbm.at[idx_v.at[pl.ds(i * C + 128, C - 128)]],
                               rows_v.at[pl.ds(128, C - 128)], gsem)
        cp0.wait()
        cp1.wait()

        def red(r, acc):
            return tuple(acc[g] + rows_v[r, pl.ds(g * 16, 16)] for g in range(4))

        zeros = (jnp.zeros((16,), jnp.float32),) * 4
        acc = lax.fori_loop(0, C, red, zeros)
        for g in range(4):
            q_v[i, pl.ds(g * 16, 16)] = acc[g] * inv_c + mrows_v[i, pl.ds(g * 16, 16)]
        return carry

    lax.fori_loop(0, NB, item, 0)
    pltpu.sync_copy(q_v, out_hbm.at[pl.ds(base, NB)])


def _build_query(ctx_flat, missing_id, table):
    mesh = plsc.VectorSubcoreMesh(core_axis_name="c", subcore_axis_name="s")
    return pl.kernel(
        _query_body,
        out_type=jax.ShapeDtypeStruct((B, D), jnp.float32),
        mesh=mesh,
        scratch_types=[
            pltpu.VMEM((IDX_PER_W,), jnp.int32),
            pltpu.VMEM((C, D), jnp.float32),
            pltpu.VMEM((NB,), jnp.int32),
            pltpu.VMEM((NB, D), jnp.float32),
            pltpu.VMEM((NB, D), jnp.float32),
            pltpu.SemaphoreType.DMA,
            pltpu.SemaphoreType.DMA,
        ],
        compiler_params=pltpu.CompilerParams(use_tc_tiling_on_sc=False),
    )(ctx_flat, missing_id, table)


TB = 8
VCHUNK = 8192


def _score_body(q_ref, t_ref, o_ref, en_ref):
    i = pl.program_id(0)

    @pl.when(i == 0)
    def _init():
        ones = jnp.ones((8, D), jnp.float32)
        n_full = V // VCHUNK
        for j in range(n_full + 1):
            w = VCHUNK if j < n_full else V - n_full * VCHUNK
            tc = t_ref[pl.ds(j * VCHUNK, w), :]
            sq = tc * tc
            en2 = lax.dot_general(ones[:, :], sq, (((1,), (1,)), ((), ())),
                                  preferred_element_type=jnp.float32)
            en_ref[:, pl.ds(j * VCHUNK, w)] = 1.0 / jnp.maximum(jnp.sqrt(en2), EPS)

    q = q_ref[...]
    qn = jnp.maximum(jnp.sqrt(jnp.sum(q * q, axis=1, keepdims=True)), EPS)
    qs = q / qn
    dots = lax.dot_general(qs, t_ref[...], (((1,), (1,)), ((), ())),
                           preferred_element_type=jnp.float32)
    o_ref[...] = dots * en_ref[0:1, :]


def kernel(context_ids, missing_id, table):
    ctx_flat = context_ids.reshape(-1).astype(jnp.int32)
    miss = missing_id.astype(jnp.int32)
    query = _build_query(ctx_flat, miss, table)
    scores = pl.pallas_call(
        _score_body,
        grid=(B // TB,),
        in_specs=[
            pl.BlockSpec((TB, D), lambda i: (i, 0)),
            pl.BlockSpec((V, D), lambda i: (0, 0)),
        ],
        out_specs=pl.BlockSpec((TB, V), lambda i: (i, 0)),
        out_shape=jax.ShapeDtypeStruct((B, V), jnp.float32),
        scratch_shapes=[pltpu.VMEM((8, V), jnp.float32)],
        compiler_params=pltpu.CompilerParams(
            vmem_limit_bytes=110 * 1024 * 1024),
    )(query, table)
    return scores


# speedup vs baseline: 2.2111x; 2.2111x over previous
import jax
import jax.numpy as jnp
from jax import lax
from jax.experimental import pallas as pl
from jax.experimental.pallas import tpu as pltpu
from jax.experimental.pallas import tpu_sc as plsc

V = 100000
D = 64
B = 1024
C = 200

NC = 2
NS = 16
NW = NC * NS
NB = B // NW
IDX_PER_W = NB * C

TV = 2048
EPS = 1e-8


def _query_body(ctx_hbm, miss_hbm, table_hbm, out_hbm,
                idx_v, rows_v, midx_v, mrows_v, q_v, gsem, msem):
    c = lax.axis_index("c")
    s = lax.axis_index("s")
    w = s * NC + c
    base = w * NB

    pltpu.sync_copy(ctx_hbm.at[pl.ds(base * C, IDX_PER_W)], idx_v)
    pltpu.sync_copy(miss_hbm.at[pl.ds(base, NB)], midx_v)
    pltpu.async_copy(table_hbm.at[midx_v], mrows_v, msem).wait()

    inv_c = jnp.float32(1.0 / C)

    def item(i, carry):
        cp0 = pltpu.async_copy(table_hbm.at[idx_v.at[pl.ds(i * C, 128)]],
                               rows_v.at[pl.ds(0, 128)], gsem)
        cp1 = pltpu.async_copy(table_hbm.at[idx_v.at[pl.ds(i * C + 128, C - 128)]],
                               rows_v.at[pl.ds(128, C - 128)], gsem)
        cp0.wait()
        cp1.wait()

        def red(r, acc):
            return tuple(acc[g] + rows_v[r, pl.ds(g * 16, 16)] for g in range(4))

        zeros = (jnp.zeros((16,), jnp.float32),) * 4
        acc = lax.fori_loop(0, C, red, zeros)
        for g in range(4):
            q_v[i, pl.ds(g * 16, 16)] = acc[g] * inv_c + mrows_v[i, pl.ds(g * 16, 16)]
        return carry

    lax.fori_loop(0, NB, item, 0)
    pltpu.sync_copy(q_v, out_hbm.at[pl.ds(base, NB)])


def _build_query(ctx_flat, missing_id, table):
    mesh = plsc.VectorSubcoreMesh(core_axis_name="c", subcore_axis_name="s")
    return pl.kernel(
        _query_body,
        out_type=jax.ShapeDtypeStruct((B, D), jnp.float32),
        mesh=mesh,
        scratch_types=[
            pltpu.VMEM((IDX_PER_W,), jnp.int32),
            pltpu.VMEM((C, D), jnp.float32),
            pltpu.VMEM((NB,), jnp.int32),
            pltpu.VMEM((NB, D), jnp.float32),
            pltpu.VMEM((NB, D), jnp.float32),
            pltpu.SemaphoreType.DMA,
            pltpu.SemaphoreType.DMA,
        ],
        compiler_params=pltpu.CompilerParams(use_tc_tiling_on_sc=False),
    )(ctx_flat, missing_id, table)


TB = 32


def _score_body(q_ref, t_ref, o_ref, en_ref):
    i = pl.program_id(0)

    @pl.when(i == 0)
    def _init():
        t = t_ref[...]
        en2 = jnp.sum(t * t, axis=0, keepdims=True)
        en_ref[...] = 1.0 / jnp.maximum(jnp.sqrt(en2), EPS)

    q = q_ref[...]
    qn = jnp.maximum(jnp.sqrt(jnp.sum(q * q, axis=1, keepdims=True)), EPS)
    qs = q / qn
    dots = lax.dot_general(qs, t_ref[...], (((1,), (0,)), ((), ())),
                           preferred_element_type=jnp.float32)
    o_ref[...] = dots * en_ref[...]


def kernel(context_ids, missing_id, table):
    ctx_flat = context_ids.reshape(-1).astype(jnp.int32)
    miss = missing_id.astype(jnp.int32)
    query = _build_query(ctx_flat, miss, table)
    table_t = table.T
    scores = pl.pallas_call(
        _score_body,
        grid=(B // TB,),
        in_specs=[
            pl.BlockSpec((TB, D), lambda i: (i, 0)),
            pl.BlockSpec((D, V), lambda i: (0, 0)),
        ],
        out_specs=pl.BlockSpec((TB, V), lambda i: (i, 0)),
        out_shape=jax.ShapeDtypeStruct((B, V), jnp.float32),
        scratch_shapes=[pltpu.VMEM((1, V), jnp.float32)],
        compiler_params=pltpu.CompilerParams(
            vmem_limit_bytes=62 * 1024 * 1024),
    )(query, table_t)
    return scores
